# baseline (speedup 1.0000x reference)
import jax
import jax.numpy as jnp
from jax import lax
from jax.experimental import pallas as pl
from jax.experimental.pallas import tpu as pltpu
from jax.experimental.pallas import tpu_sc as plsc

B = 4096
L = 200
D = 64
NC = 2
NS = 16
NW = NC * NS
ROWS = B * L
RPW = ROWS // NW
BW = 256
CPP = B // BW
NCHUNK = RPW // BW
G = 128
NG = BW // G
NBUF = 2
LANES = 16
VPR = D // LANES
VROWS = 1000001
PADV = 1000008
TROWS = PADV + L


def _sc_body(seq_hbm, tab_hbm, out_hbm, posv, idxall, rowsv, outv, *sems):
    gsems = sems[:NBUF]
    osems = sems[NBUF:]
    cid = lax.axis_index("c")
    sid = lax.axis_index("s")
    wid = sid * NC + cid

    pltpu.sync_copy(tab_hbm.at[pl.ds(PADV, L)], posv)
    pltpu.sync_copy(seq_hbm.at[pl.ds(wid * RPW, RPW)], idxall)

    iota = lax.iota(jnp.int32, LANES)

    def gathers_start(c, b):
        for h in range(NG):
            pltpu.async_copy(
                tab_hbm.at[idxall.at[pl.ds((c * NG + h) * G, G)]],
                rowsv.at[pl.ds(b * BW + h * G, G)],
                gsems[b],
            )

    def gathers_wait(c, b):
        for h in range(NG):
            pltpu.make_async_copy(
                tab_hbm.at[idxall.at[pl.ds((c * NG + h) * G, G)]],
                rowsv.at[pl.ds(b * BW + h * G, G)],
                gsems[b],
            ).wait()

    def out_copy(c, b, start):
        gc = wid * NCHUNK + c
        l = gc // CPP
        b0 = (gc % CPP) * BW
        desc = pltpu.make_async_copy(
            outv.at[pl.ds(b * D, D)],
            out_hbm.at[l, :, pl.ds(b0, BW)],
            osems[b],
        )
        if start:
            desc.start()
        else:
            desc.wait()

    for b in range(NBUF - 1):
        gathers_start(b, b)

    @pl.loop(0, NCHUNK, step=NBUF)
    def outer(C):
        for b in range(NBUF):
            c = C + b
            bg = (b + NBUF - 1) % NBUF
            g = c + NBUF - 1

            gathers_wait(c, b)

            gc = wid * NCHUNK + c
            l = gc // CPP
            pos = [posv[l, pl.ds(k * LANES, LANES)] for k in range(VPR)]
            rowk = [
                jnp.full((LANES,), b * D + k * LANES, jnp.int32) + iota
                for k in range(VPR)
            ]
            rbase = b * BW

            @pl.loop(0, BW)
            def row_loop(j):
                colj = jnp.full((LANES,), j, jnp.int32)
                for k in range(VPR):
                    v = rowsv[rbase + j, pl.ds(k * LANES, LANES)] + pos[k]
                    plsc.store_scatter(outv, [rowk[k], colj], v)

            out_copy(c, b, start=True)

            @pl.when(g < NCHUNK)
            def _():
                @pl.when(c >= 1)
                def _():
                    out_copy(c - 1, bg, start=False)

                gathers_start(g, bg)

    for b in range(NBUF):
        out_copy(NCHUNK - NBUF + b, b, start=False)


@jax.jit
def _sc_call(seq_flat, tab):
    mesh = plsc.VectorSubcoreMesh(
        core_axis_name="c", subcore_axis_name="s", num_cores=NC, num_subcores=NS
    )
    return pl.kernel(
        _sc_body,
        out_type=jax.ShapeDtypeStruct((L, D, B), jnp.float32),
        mesh=mesh,
        compiler_params=pltpu.CompilerParams(
            use_tc_tiling_on_sc=False, needs_layout_passes=False
        ),
        scratch_types=[
            pltpu.VMEM((L, D), jnp.float32),
            pltpu.VMEM((RPW,), jnp.int32),
            pltpu.VMEM((NBUF * BW, D), jnp.float32),
            pltpu.VMEM((NBUF * D, BW), jnp.float32),
        ]
        + [pltpu.SemaphoreType.DMA] * (2 * NBUF),
    )(seq_flat, tab)


def kernel(seq, pos, neg, item_embedding, position_embedding):
    del pos, neg
    seq_flat = seq.T.astype(jnp.int32).reshape(-1)
    tab = jnp.concatenate(
        [
            item_embedding,
            jnp.zeros((PADV - VROWS, D), jnp.float32),
            position_embedding,
        ],
        axis=0,
    )
    out = _sc_call(seq_flat, tab)
    return jnp.transpose(out, (2, 0, 1))


# speedup vs baseline: 1.1513x; 1.1513x over previous
import jax
import jax.numpy as jnp
from jax import lax
from jax.experimental import pallas as pl
from jax.experimental.pallas import tpu as pltpu
from jax.experimental.pallas import tpu_sc as plsc

B = 4096
L = 200
D = 64
NC = 2
NS = 16
NW = NC * NS
ROWS = B * L
RPW = ROWS // NW
BW = 256
CPP = B // BW
NCHUNK = RPW // BW
G = 128
NG = BW // G
NBUF = 2
LANES = 16
VPR = D // LANES
VROWS = 1000001
PADV = 1000008
TROWS = PADV + L


def _sc_body(seq_hbm, tab_hbm, out_hbm, posv, idxall, rowsv, outv, *sems):
    gsems = sems[:NBUF]
    osems = sems[NBUF:]
    cid = lax.axis_index("c")
    sid = lax.axis_index("s")
    wid = sid * NC + cid

    pltpu.sync_copy(tab_hbm.at[pl.ds(PADV, L)], posv)
    pltpu.sync_copy(seq_hbm.at[pl.ds(wid * RPW, RPW)], idxall)

    iota = lax.iota(jnp.int32, LANES)

    def gathers_start(c, b):
        for h in range(NG):
            pltpu.async_copy(
                tab_hbm.at[idxall.at[pl.ds((c * NG + h) * G, G)]],
                rowsv.at[pl.ds(b * BW + h * G, G)],
                gsems[b],
            )

    def gathers_wait(c, b):
        for h in range(NG):
            pltpu.make_async_copy(
                tab_hbm.at[idxall.at[pl.ds((c * NG + h) * G, G)]],
                rowsv.at[pl.ds(b * BW + h * G, G)],
                gsems[b],
            ).wait()

    def out_copy(c, b, start):
        gc = wid * NCHUNK + c
        l = gc // CPP
        b0 = (gc % CPP) * BW
        desc = pltpu.make_async_copy(
            outv.at[pl.ds(b * D, D)],
            out_hbm.at[l, :, pl.ds(b0, BW)],
            osems[b],
        )
        if start:
            desc.start()
        else:
            desc.wait()

    for b in range(NBUF - 1):
        gathers_start(b, b)

    @pl.loop(0, NCHUNK, step=NBUF)
    def outer(C):
        for b in range(NBUF):
            c = C + b
            bg = (b + NBUF - 1) % NBUF
            g = c + NBUF - 1

            gathers_wait(c, b)

            gc = wid * NCHUNK + c
            l = gc // CPP
            pos = [posv[l, pl.ds(k * LANES, LANES)] for k in range(VPR)]
            rowk = [
                jnp.full((LANES,), b * D + k * LANES, jnp.int32) + iota
                for k in range(VPR)
            ]
            rbase = b * BW

            @plsc.parallel_loop(0, BW, unroll=4)
            def row_loop(j):
                colj = jnp.full((LANES,), j, jnp.int32)
                for k in range(VPR):
                    v = rowsv[rbase + j, pl.ds(k * LANES, LANES)] + pos[k]
                    plsc.store_scatter(outv, [rowk[k], colj], v)

            out_copy(c, b, start=True)

            @pl.when(g < NCHUNK)
            def _():
                @pl.when(c >= 1)
                def _():
                    out_copy(c - 1, bg, start=False)

                gathers_start(g, bg)

    for b in range(NBUF):
        out_copy(NCHUNK - NBUF + b, b, start=False)


@jax.jit
def _sc_call(seq_flat, tab):
    mesh = plsc.VectorSubcoreMesh(
        core_axis_name="c", subcore_axis_name="s", num_cores=NC, num_subcores=NS
    )
    return pl.kernel(
        _sc_body,
        out_type=jax.ShapeDtypeStruct((L, D, B), jnp.float32),
        mesh=mesh,
        compiler_params=pltpu.CompilerParams(
            use_tc_tiling_on_sc=False, needs_layout_passes=False
        ),
        scratch_types=[
            pltpu.VMEM((L, D), jnp.float32),
            pltpu.VMEM((RPW,), jnp.int32),
            pltpu.VMEM((NBUF * BW, D), jnp.float32),
            pltpu.VMEM((NBUF * D, BW), jnp.float32),
        ]
        + [pltpu.SemaphoreType.DMA] * (2 * NBUF),
    )(seq_flat, tab)


def kernel(seq, pos, neg, item_embedding, position_embedding):
    del pos, neg
    seq_flat = seq.T.astype(jnp.int32).reshape(-1)
    tab = jnp.concatenate(
        [
            item_embedding,
            jnp.zeros((PADV - VROWS, D), jnp.float32),
            position_embedding,
        ],
        axis=0,
    )
    out = _sc_call(seq_flat, tab)
    return jnp.transpose(out, (2, 0, 1))


# speedup vs baseline: 2.5264x; 2.1943x over previous
import jax
import jax.numpy as jnp
from jax import lax
from jax.experimental import pallas as pl
from jax.experimental.pallas import tpu as pltpu
from jax.experimental.pallas import tpu_sc as plsc

B = 4096
L = 200
D = 64
NC = 2
NS = 16
NW = NC * NS
ROWS = B * L
RPW = ROWS // NW
BW = 256
CPP = B // BW
NCHUNK = RPW // BW
G = 128
NG = BW // G
NBUF = 2
LANES = 16
VPR = D // LANES


def _sc_body(seq_hbm, tab_hbm, pos_hbm, out_hbm, posv, idxall, rowsv, *sems):
    gsems = sems[:NBUF]
    osems = sems[NBUF:]
    cid = lax.axis_index("c")
    sid = lax.axis_index("s")
    wid = sid * NC + cid

    pltpu.sync_copy(pos_hbm, posv)
    pltpu.sync_copy(seq_hbm.at[pl.ds(wid * RPW, RPW)], idxall)

    def gathers_start(c, b):
        for h in range(NG):
            pltpu.async_copy(
                tab_hbm.at[idxall.at[pl.ds((c * NG + h) * G, G)]],
                rowsv.at[pl.ds(b * BW + h * G, G)],
                gsems[b],
            )

    def gathers_wait(c, b):
        for h in range(NG):
            pltpu.make_async_copy(
                tab_hbm.at[idxall.at[pl.ds((c * NG + h) * G, G)]],
                rowsv.at[pl.ds(b * BW + h * G, G)],
                gsems[b],
            ).wait()

    def out_copy(c, b, start):
        gc = wid * NCHUNK + c
        desc = pltpu.make_async_copy(
            rowsv.at[pl.ds(b * BW, BW)],
            out_hbm.at[pl.ds(gc * BW, BW)],
            osems[b],
        )
        if start:
            desc.start()
        else:
            desc.wait()

    for b in range(NBUF - 1):
        gathers_start(b, b)

    @pl.loop(0, NCHUNK, step=NBUF)
    def outer(C):
        for b in range(NBUF):
            c = C + b
            bg = (b + NBUF - 1) % NBUF
            g = c + NBUF - 1

            gathers_wait(c, b)

            gc = wid * NCHUNK + c
            l = gc // CPP
            pos = [posv[l, pl.ds(k * LANES, LANES)] for k in range(VPR)]
            rbase = b * BW

            @plsc.parallel_loop(0, BW, unroll=8)
            def row_loop(j):
                for k in range(VPR):
                    sl = pl.ds(k * LANES, LANES)
                    rowsv[rbase + j, sl] = rowsv[rbase + j, sl] + pos[k]

            out_copy(c, b, start=True)

            @pl.when(g < NCHUNK)
            def _():
                @pl.when(c >= 1)
                def _():
                    out_copy(c - 1, bg, start=False)

                gathers_start(g, bg)

    for b in range(NBUF):
        out_copy(NCHUNK - NBUF + b, b, start=False)


@jax.jit
def _sc_call(seq_flat, tab, pos_tab):
    mesh = plsc.VectorSubcoreMesh(
        core_axis_name="c", subcore_axis_name="s", num_cores=NC, num_subcores=NS
    )
    return pl.kernel(
        _sc_body,
        out_type=jax.ShapeDtypeStruct((ROWS, D), jnp.float32),
        mesh=mesh,
        compiler_params=pltpu.CompilerParams(
            use_tc_tiling_on_sc=False, needs_layout_passes=False
        ),
        scratch_types=[
            pltpu.VMEM((L, D), jnp.float32),
            pltpu.VMEM((RPW,), jnp.int32),
            pltpu.VMEM((NBUF * BW, D), jnp.float32),
        ]
        + [pltpu.SemaphoreType.DMA] * (2 * NBUF),
    )(seq_flat, tab, pos_tab)


def kernel(seq, pos, neg, item_embedding, position_embedding):
    del pos, neg
    seq_flat = seq.T.astype(jnp.int32).reshape(-1)
    out = _sc_call(seq_flat, item_embedding, position_embedding)
    return jnp.transpose(out.reshape(L, B, D), (1, 0, 2))


# speedup vs baseline: 2.6153x; 1.0352x over previous
import jax
import jax.numpy as jnp
from jax import lax
from jax.experimental import pallas as pl
from jax.experimental.pallas import tpu as pltpu
from jax.experimental.pallas import tpu_sc as plsc

B = 4096
L = 200
D = 64
NC = 2
NS = 16
NW = NC * NS
ROWS = B * L
RPW = ROWS // NW
BW = 256
CPP = B // BW
NCHUNK = RPW // BW
G = 128
NG = BW // G
NBUF = 2
LANES = 16
VPR = D // LANES


def _sc_body(seq_hbm, tab_hbm, pos_hbm, out_hbm, posv, idxall, rowsv, *sems):
    gsems = sems[:NBUF]
    osems = sems[NBUF:]
    cid = lax.axis_index("c")
    sid = lax.axis_index("s")
    wid = sid * NC + cid

    pltpu.sync_copy(pos_hbm, posv)
    pltpu.sync_copy(seq_hbm.at[pl.ds(wid * RPW, RPW)], idxall)

    def gathers_start(c, b):
        for h in range(NG):
            pltpu.async_copy(
                tab_hbm.at[idxall.at[pl.ds((c * NG + h) * G, G)]],
                rowsv.at[pl.ds(b * BW + h * G, G)],
                gsems[b],
            )

    def gathers_wait(c, b):
        for h in range(NG):
            pltpu.make_async_copy(
                tab_hbm.at[idxall.at[pl.ds((c * NG + h) * G, G)]],
                rowsv.at[pl.ds(b * BW + h * G, G)],
                gsems[b],
            ).wait()

    def out_copy(c, b, start):
        gc = wid * NCHUNK + c
        l = gc // CPP
        b0 = (gc % CPP) * BW
        desc = pltpu.make_async_copy(
            rowsv.at[pl.ds(b * BW, BW)],
            out_hbm.at[l, pl.ds(b0, BW), pl.ds(0, D)],
            osems[b],
        )
        if start:
            desc.start()
        else:
            desc.wait()

    for b in range(NBUF - 1):
        gathers_start(b, b)

    @pl.loop(0, NCHUNK, step=NBUF)
    def outer(C):
        for b in range(NBUF):
            c = C + b
            bg = (b + NBUF - 1) % NBUF
            g = c + NBUF - 1

            gathers_wait(c, b)

            gc = wid * NCHUNK + c
            l = gc // CPP
            pos = [posv[l, pl.ds(k * LANES, LANES)] for k in range(VPR)]
            rbase = b * BW

            @plsc.parallel_loop(0, BW, unroll=8)
            def row_loop(j):
                for k in range(VPR):
                    sl = pl.ds(k * LANES, LANES)
                    rowsv[rbase + j, sl] = rowsv[rbase + j, sl] + pos[k]

            out_copy(c, b, start=True)

            @pl.when(g < NCHUNK)
            def _():
                @pl.when(c >= 1)
                def _():
                    out_copy(c - 1, bg, start=False)

                gathers_start(g, bg)

    for b in range(NBUF):
        out_copy(NCHUNK - NBUF + b, b, start=False)


@jax.jit
def _sc_call(seq_flat, tab, pos_tab):
    mesh = plsc.VectorSubcoreMesh(
        core_axis_name="c", subcore_axis_name="s", num_cores=NC, num_subcores=NS
    )
    return pl.kernel(
        _sc_body,
        out_type=jax.ShapeDtypeStruct((L, B, 2 * D), jnp.float32),
        mesh=mesh,
        compiler_params=pltpu.CompilerParams(
            use_tc_tiling_on_sc=False, needs_layout_passes=False
        ),
        scratch_types=[
            pltpu.VMEM((L, D), jnp.float32),
            pltpu.VMEM((RPW,), jnp.int32),
            pltpu.VMEM((NBUF * BW, D), jnp.float32),
        ]
        + [pltpu.SemaphoreType.DMA] * (2 * NBUF),
    )(seq_flat, tab, pos_tab)


def kernel(seq, pos, neg, item_embedding, position_embedding):
    del pos, neg
    seq_flat = seq.T.astype(jnp.int32).reshape(-1)
    out = _sc_call(seq_flat, item_embedding, position_embedding)
    return jnp.transpose(out[:, :, :D], (1, 0, 2))
